# Initial kernel scaffold; baseline (speedup 1.0000x reference)
import jax
import jax.numpy as jnp
from jax.experimental import pallas as pl


def kernel(prev, raw, edge_index, W, b):
    raise NotImplementedError("write your pallas kernel here")


# speedup vs baseline: 4.7950x; 4.7950x over previous
import functools

import jax
import jax.numpy as jnp
from jax import lax
from jax.experimental import pallas as pl
from jax.experimental.pallas import tpu as pltpu
from jax.experimental.pallas import tpu_sc as plsc

N = 10000
E = 160000
D = 256
HD = D // 2
N2 = 10240
RT = N2 // 16
CH = 80
CPT = E // (16 * CH)
BR = 1280

_mesh = plsc.VectorSubcoreMesh(core_axis_name="c", subcore_axis_name="s")


@functools.partial(
    pl.kernel,
    out_type=(
        jax.ShapeDtypeStruct((N2,), jnp.float32),
        jax.ShapeDtypeStruct((N2,), jnp.float32),
    ),
    mesh=_mesh,
    scratch_types=[
        pltpu.VMEM((CH,), jnp.int32),
        pltpu.VMEM((CH,), jnp.float32),
        pltpu.VMEM_SHARED((N2,), jnp.float32),
    ],
)
def _degrees(dst_hbm, src_hbm, zvec_hbm, din_hbm, dout_hbm, idx_v, ones_v, acc_sh):
    c = lax.axis_index("c")
    s = lax.axis_index("s")
    for i in range(CH // 16):
        ones_v[pl.ds(i * 16, 16)] = jnp.full((16,), 1.0, jnp.float32)
    pltpu.sync_copy(zvec_hbm, acc_sh.at[pl.ds(s * RT, RT)])
    plsc.subcore_barrier()

    def chunk(k, idx_hbm):
        e0 = (s * CPT + k) * CH
        pltpu.sync_copy(idx_hbm.at[pl.ds(e0, CH)], idx_v)
        pltpu.sync_copy(ones_v, acc_sh.at[idx_v], add=True)

    @pl.when(c == 0)
    def _():
        lax.fori_loop(0, CPT, lambda k, u: (chunk(k, dst_hbm), u)[1], 0)

    @pl.when(c == 1)
    def _():
        lax.fori_loop(0, CPT, lambda k, u: (chunk(k, src_hbm), u)[1], 0)

    plsc.subcore_barrier()

    @pl.when(c == 0)
    def _():
        pltpu.sync_copy(acc_sh.at[pl.ds(s * RT, RT)], din_hbm.at[pl.ds(s * RT, RT)])

    @pl.when(c == 1)
    def _():
        pltpu.sync_copy(acc_sh.at[pl.ds(s * RT, RT)], dout_hbm.at[pl.ds(s * RT, RT)])


@functools.partial(
    pl.kernel,
    out_type=(
        jax.ShapeDtypeStruct((N2, HD), jnp.float32),
        jax.ShapeDtypeStruct((N2, HD), jnp.float32),
    ),
    mesh=_mesh,
    scratch_types=[
        pltpu.VMEM((CH,), jnp.int32),
        pltpu.VMEM((CH,), jnp.int32),
        pltpu.VMEM((CH,), jnp.int32),
        pltpu.VMEM((CH, HD), jnp.float32),
        pltpu.VMEM_SHARED((N2, HD), jnp.float32),
        pltpu.SemaphoreType.DMA,
    ],
)
def _aggregate(x2_hbm, src_hbm, dst_hbm, zrows_hbm, h0_hbm, h1_hbm,
               src_v, dst_v, gidx_v, rows_v, acc_sh, sem):
    c = lax.axis_index("c")
    s = lax.axis_index("s")
    pltpu.sync_copy(zrows_hbm, acc_sh.at[pl.ds(s * RT, RT)])
    plsc.subcore_barrier()
    cvec = jnp.zeros((16,), jnp.int32) + c

    def chunk(k, u):
        e0 = (s * CPT + k) * CH
        pltpu.sync_copy(src_hbm.at[pl.ds(e0, CH)], src_v)
        pltpu.sync_copy(dst_hbm.at[pl.ds(e0, CH)], dst_v)
        for i in range(CH // 16):
            v = src_v[pl.ds(i * 16, 16)]
            gidx_v[pl.ds(i * 16, 16)] = v + v + cvec
        pltpu.async_copy(x2_hbm.at[gidx_v], rows_v, sem).wait()
        pltpu.sync_copy(rows_v, acc_sh.at[dst_v], add=True)
        return u

    lax.fori_loop(0, CPT, chunk, 0)
    plsc.subcore_barrier()

    @pl.when(c == 0)
    def _():
        pltpu.sync_copy(acc_sh.at[pl.ds(s * RT, RT)], h0_hbm.at[pl.ds(s * RT, RT)])

    @pl.when(c == 1)
    def _():
        pltpu.sync_copy(acc_sh.at[pl.ds(s * RT, RT)], h1_hbm.at[pl.ds(s * RT, RT)])


def _dense_body(prev_ref, raw_ref, w_ref, din_ref, dout_ref, x_ref):
    n_in = lax.rsqrt(jnp.maximum(din_ref[...], 1.0))
    n_out = lax.rsqrt(jnp.maximum(dout_ref[...], 1.0))
    p = jnp.dot(prev_ref[...], w_ref[...], preferred_element_type=jnp.float32)
    x_ref[...] = raw_ref[...] * n_in + p * n_out


def _dense(prev, raw, W, din2, dout2):
    return pl.pallas_call(
        _dense_body,
        grid=(N2 // BR,),
        in_specs=[
            pl.BlockSpec((BR, D), lambda i: (i, 0)),
            pl.BlockSpec((BR, D), lambda i: (i, 0)),
            pl.BlockSpec((D, D), lambda i: (0, 0)),
            pl.BlockSpec((BR, 1), lambda i: (i, 0)),
            pl.BlockSpec((BR, 1), lambda i: (i, 0)),
        ],
        out_specs=pl.BlockSpec((BR, D), lambda i: (i, 0)),
        out_shape=jax.ShapeDtypeStruct((N2, D), jnp.float32),
    )(prev, raw, W, din2, dout2)


def _final_body(h0_ref, h1_ref, din_ref, b_ref, o_ref):
    n_in = lax.rsqrt(jnp.maximum(din_ref[...], 1.0))
    h = jnp.concatenate([h0_ref[...], h1_ref[...]], axis=1)
    o_ref[...] = jnp.maximum(h * n_in + b_ref[...], 0.0)


def _final(h0, h1, din2, b2):
    return pl.pallas_call(
        _final_body,
        grid=(N2 // BR,),
        in_specs=[
            pl.BlockSpec((BR, HD), lambda i: (i, 0)),
            pl.BlockSpec((BR, HD), lambda i: (i, 0)),
            pl.BlockSpec((BR, 1), lambda i: (i, 0)),
            pl.BlockSpec((1, D), lambda i: (0, 0)),
        ],
        out_specs=pl.BlockSpec((BR, D), lambda i: (i, 0)),
        out_shape=jax.ShapeDtypeStruct((N, D), jnp.float32),
    )(h0, h1, din2, b2)


def kernel(prev, raw, edge_index, W, b):
    src = edge_index[0]
    dst = edge_index[1]
    zvec = jnp.zeros((RT,), jnp.float32)
    zrows = jnp.zeros((RT, HD), jnp.float32)

    deg_in, deg_out = _degrees(dst, src, zvec)
    din2 = deg_in.reshape(N2, 1)
    dout2 = deg_out.reshape(N2, 1)

    x = _dense(prev, raw, W, din2, dout2)
    x2 = x.reshape(2 * N2, HD)

    h0, h1 = _aggregate(x2, src, dst, zrows)
    return _final(h0, h1, din2, b.reshape(1, D))


# speedup vs baseline: 10.5314x; 2.1963x over previous
import functools

import jax
import jax.numpy as jnp
from jax import lax
from jax.experimental import pallas as pl
from jax.experimental.pallas import tpu as pltpu
from jax.experimental.pallas import tpu_sc as plsc

N = 10000
E = 160000
D = 256
HD = D // 2
N2 = 10240
RT = N2 // 16
CH = 128
CPT = 80
E2 = 16 * CPT * CH
EPT = CPT * CH
ER = E2 // CH
BR = 1280

_mesh = plsc.VectorSubcoreMesh(core_axis_name="c", subcore_axis_name="s")


@functools.partial(
    pl.kernel,
    out_type=(
        jax.ShapeDtypeStruct((N2,), jnp.float32),
        jax.ShapeDtypeStruct((N2,), jnp.float32),
    ),
    mesh=_mesh,
    scratch_types=[
        pltpu.VMEM((CPT, CH), jnp.int32),
        pltpu.VMEM((CH,), jnp.float32),
        pltpu.VMEM_SHARED((N2,), jnp.float32),
    ],
)
def _degrees(dst_hbm, src_hbm, zvec_hbm, din_hbm, dout_hbm, idx_all, ones_v, acc_sh):
    c = lax.axis_index("c")
    s = lax.axis_index("s")
    for i in range(CH // 16):
        ones_v[pl.ds(i * 16, 16)] = jnp.full((16,), 1.0, jnp.float32)
    pltpu.sync_copy(zvec_hbm, acc_sh.at[pl.ds(s * RT, RT)])

    @pl.when(c == 0)
    def _():
        pltpu.sync_copy(dst_hbm.at[pl.ds(s * CPT, CPT)], idx_all)

    @pl.when(c == 1)
    def _():
        pltpu.sync_copy(src_hbm.at[pl.ds(s * CPT, CPT)], idx_all)

    plsc.subcore_barrier()

    def chunk(k, u):
        pltpu.sync_copy(ones_v, acc_sh.at[idx_all.at[k]], add=True)
        return u

    lax.fori_loop(0, CPT, chunk, 0)
    plsc.subcore_barrier()

    @pl.when(c == 0)
    def _():
        pltpu.sync_copy(acc_sh.at[pl.ds(s * RT, RT)], din_hbm.at[pl.ds(s * RT, RT)])

    @pl.when(c == 1)
    def _():
        pltpu.sync_copy(acc_sh.at[pl.ds(s * RT, RT)], dout_hbm.at[pl.ds(s * RT, RT)])


@functools.partial(
    pl.kernel,
    out_type=(
        jax.ShapeDtypeStruct((N2, HD), jnp.float32),
        jax.ShapeDtypeStruct((N2, HD), jnp.float32),
    ),
    mesh=_mesh,
    scratch_types=[
        pltpu.VMEM((2, CH), jnp.int32),
        pltpu.VMEM((CPT, CH), jnp.int32),
        pltpu.VMEM((CH, HD), jnp.float32),
        pltpu.VMEM((CH, HD), jnp.float32),
        pltpu.VMEM_SHARED((N2, HD), jnp.float32),
        pltpu.SemaphoreType.DMA,
        pltpu.SemaphoreType.DMA,
        pltpu.SemaphoreType.DMA,
        pltpu.SemaphoreType.DMA,
    ],
)
def _aggregate(xlo_hbm, xhi_hbm, srcf_hbm, dst_hbm, zrows_hbm, h0_hbm, h1_hbm,
               src_v, dst_all, rows_a, rows_b, acc_sh, gsem_a, gsem_b,
               isem_a, isem_b):
    c = lax.axis_index("c")
    s = lax.axis_index("s")
    pltpu.sync_copy(zrows_hbm, acc_sh.at[pl.ds(s * RT, RT)])
    pltpu.sync_copy(dst_hbm.at[pl.ds(s * CPT, CPT)], dst_all)
    plsc.subcore_barrier()

    def run(x_hbm):
        def idx_start(k, slot, isem):
            pltpu.async_copy(srcf_hbm.at[pl.ds((s * CPT + k) * CH, CH)],
                             src_v.at[slot], isem)

        def idx_wait(k, slot, isem):
            pltpu.make_async_copy(srcf_hbm.at[pl.ds((s * CPT + k) * CH, CH)],
                                  src_v.at[slot], isem).wait()

        def gather_start(slot, rows, gsem):
            pltpu.async_copy(x_hbm.at[src_v.at[slot]], rows, gsem)

        def gather_wait(slot, rows, gsem):
            pltpu.make_async_copy(x_hbm.at[src_v.at[slot]], rows, gsem).wait()

        def scatter(k, rows):
            pltpu.sync_copy(rows, acc_sh.at[dst_all.at[k]], add=True)

        idx_start(0, 0, isem_a)
        idx_wait(0, 0, isem_a)
        gather_start(0, rows_a, gsem_a)
        idx_start(1, 1, isem_b)

        def pair(j, u):
            k0 = 2 * j
            gather_wait(0, rows_a, gsem_a)
            idx_wait(k0 + 1, 1, isem_b)
            gather_start(1, rows_b, gsem_b)

            @pl.when(k0 + 2 < CPT)
            def _():
                idx_start(k0 + 2, 0, isem_a)

            scatter(k0, rows_a)
            gather_wait(1, rows_b, gsem_b)

            @pl.when(k0 + 2 < CPT)
            def _():
                idx_wait(k0 + 2, 0, isem_a)
                gather_start(0, rows_a, gsem_a)

            @pl.when(k0 + 3 < CPT)
            def _():
                idx_start(k0 + 3, 1, isem_b)

            scatter(k0 + 1, rows_b)
            return u

        lax.fori_loop(0, CPT // 2, pair, 0)

    @pl.when(c == 0)
    def _():
        run(xlo_hbm)

    @pl.when(c == 1)
    def _():
        run(xhi_hbm)

    plsc.subcore_barrier()

    @pl.when(c == 0)
    def _():
        pltpu.sync_copy(acc_sh.at[pl.ds(s * RT, RT)], h0_hbm.at[pl.ds(s * RT, RT)])

    @pl.when(c == 1)
    def _():
        pltpu.sync_copy(acc_sh.at[pl.ds(s * RT, RT)], h1_hbm.at[pl.ds(s * RT, RT)])


def _dense_body(prev_ref, raw_ref, w_ref, din_ref, dout_ref, xlo_ref, xhi_ref):
    n_in = lax.rsqrt(jnp.maximum(din_ref[...], 1.0))
    n_out = lax.rsqrt(jnp.maximum(dout_ref[...], 1.0))
    p = jnp.dot(prev_ref[...], w_ref[...], preferred_element_type=jnp.float32)
    x = raw_ref[...] * n_in + p * n_out
    xlo_ref[...] = x[:, :HD]
    xhi_ref[...] = x[:, HD:]


def _dense(prev, raw, W, din2, dout2):
    return pl.pallas_call(
        _dense_body,
        grid=(N2 // BR,),
        in_specs=[
            pl.BlockSpec((BR, D), lambda i: (i, 0)),
            pl.BlockSpec((BR, D), lambda i: (i, 0)),
            pl.BlockSpec((D, D), lambda i: (0, 0)),
            pl.BlockSpec((BR, 1), lambda i: (i, 0)),
            pl.BlockSpec((BR, 1), lambda i: (i, 0)),
        ],
        out_specs=(
            pl.BlockSpec((BR, HD), lambda i: (i, 0)),
            pl.BlockSpec((BR, HD), lambda i: (i, 0)),
        ),
        out_shape=(
            jax.ShapeDtypeStruct((N2, HD), jnp.float32),
            jax.ShapeDtypeStruct((N2, HD), jnp.float32),
        ),
    )(prev, raw, W, din2, dout2)


def _final_body(h0_ref, h1_ref, din_ref, b_ref, o_ref):
    n_in = lax.rsqrt(jnp.maximum(din_ref[...], 1.0))
    h = jnp.concatenate([h0_ref[...], h1_ref[...]], axis=1)
    o_ref[...] = jnp.maximum(h * n_in + b_ref[...], 0.0)


def _final(h0, h1, din2, b2):
    return pl.pallas_call(
        _final_body,
        grid=(N2 // BR,),
        in_specs=[
            pl.BlockSpec((BR, HD), lambda i: (i, 0)),
            pl.BlockSpec((BR, HD), lambda i: (i, 0)),
            pl.BlockSpec((BR, 1), lambda i: (i, 0)),
            pl.BlockSpec((1, D), lambda i: (0, 0)),
        ],
        out_specs=pl.BlockSpec((BR, D), lambda i: (i, 0)),
        out_shape=jax.ShapeDtypeStruct((N, D), jnp.float32),
    )(h0, h1, din2, b2)


def kernel(prev, raw, edge_index, W, b):
    npad = E2 - E
    pad_rows = N + (jnp.arange(npad, dtype=jnp.int32) % (N2 - N))
    srcf = jnp.concatenate([edge_index[0], pad_rows])
    src2d = srcf.reshape(ER, CH)
    dst2d = jnp.concatenate([edge_index[1], pad_rows]).reshape(ER, CH)
    zvec = jnp.zeros((RT,), jnp.float32)
    zrows = jnp.zeros((RT, HD), jnp.float32)

    deg_in, deg_out = _degrees(dst2d, src2d, zvec)
    din2 = deg_in.reshape(N2, 1)
    dout2 = deg_out.reshape(N2, 1)

    xlo, xhi = _dense(prev, raw, W, din2, dout2)
    h0, h1 = _aggregate(xlo, xhi, srcf, dst2d, zrows)
    return _final(h0, h1, din2, b.reshape(1, D))


# speedup vs baseline: 11.2357x; 1.0669x over previous
import functools

import jax
import jax.numpy as jnp
from jax import lax
from jax.experimental import pallas as pl
from jax.experimental.pallas import tpu as pltpu
from jax.experimental.pallas import tpu_sc as plsc

N = 10000
E = 160000
D = 256
HD = D // 2
N2 = 10240
RT = N2 // 16
NA = 10112
RTA = NA // 16
CH = 128
CPT = 80
E2 = 16 * CPT * CH
EPT = CPT * CH
ER = E2 // CH
BR = N2 // 8
BRF = NA // 8

_mesh = plsc.VectorSubcoreMesh(core_axis_name="c", subcore_axis_name="s")


@functools.partial(
    pl.kernel,
    out_type=(
        jax.ShapeDtypeStruct((N2,), jnp.float32),
        jax.ShapeDtypeStruct((N2,), jnp.float32),
    ),
    mesh=_mesh,
    scratch_types=[
        pltpu.VMEM((CPT, CH), jnp.int32),
        pltpu.VMEM((CH,), jnp.float32),
        pltpu.VMEM_SHARED((N2,), jnp.float32),
    ],
)
def _degrees(dst_hbm, src_hbm, zvec_hbm, din_hbm, dout_hbm, idx_all, ones_v, acc_sh):
    c = lax.axis_index("c")
    s = lax.axis_index("s")
    for i in range(CH // 16):
        ones_v[pl.ds(i * 16, 16)] = jnp.full((16,), 1.0, jnp.float32)
    pltpu.sync_copy(zvec_hbm, acc_sh.at[pl.ds(s * RT, RT)])

    @pl.when(c == 0)
    def _():
        pltpu.sync_copy(dst_hbm.at[pl.ds(s * CPT, CPT)], idx_all)

    @pl.when(c == 1)
    def _():
        pltpu.sync_copy(src_hbm.at[pl.ds(s * CPT, CPT)], idx_all)

    plsc.subcore_barrier()

    def chunk(k, u):
        pltpu.sync_copy(ones_v, acc_sh.at[idx_all.at[k]], add=True)
        return u

    lax.fori_loop(0, CPT, chunk, 0)
    plsc.subcore_barrier()

    @pl.when(c == 0)
    def _():
        pltpu.sync_copy(acc_sh.at[pl.ds(s * RT, RT)], din_hbm.at[pl.ds(s * RT, RT)])

    @pl.when(c == 1)
    def _():
        pltpu.sync_copy(acc_sh.at[pl.ds(s * RT, RT)], dout_hbm.at[pl.ds(s * RT, RT)])


@functools.partial(
    pl.kernel,
    out_type=(
        jax.ShapeDtypeStruct((NA, HD), jnp.float32),
        jax.ShapeDtypeStruct((NA, HD), jnp.float32),
    ),
    mesh=_mesh,
    scratch_types=[
        pltpu.VMEM((3, CH), jnp.int32),
        pltpu.VMEM((3, CH), jnp.int32),
        pltpu.VMEM((CH, HD), jnp.float32),
        pltpu.VMEM((CH, HD), jnp.float32),
        pltpu.VMEM((CH, HD), jnp.float32),
        pltpu.VMEM_SHARED((NA, HD), jnp.float32),
        pltpu.SemaphoreType.DMA,
        pltpu.SemaphoreType.DMA,
        pltpu.SemaphoreType.DMA,
        pltpu.SemaphoreType.DMA,
        pltpu.SemaphoreType.DMA,
        pltpu.SemaphoreType.DMA,
    ],
)
def _aggregate(xlo_hbm, xhi_hbm, srcf_hbm, dstf_hbm, zrows_hbm, h0_hbm, h1_hbm,
               src_v, dst_v, rows_0, rows_1, rows_2, acc_sh,
               gsem_0, gsem_1, gsem_2, isem_0, isem_1, isem_2):
    c = lax.axis_index("c")
    s = lax.axis_index("s")
    pltpu.sync_copy(zrows_hbm, acc_sh.at[pl.ds(s * RTA, RTA)])
    plsc.subcore_barrier()
    rows = (rows_0, rows_1, rows_2)
    gsem = (gsem_0, gsem_1, gsem_2)
    isem = (isem_0, isem_1, isem_2)

    def run(x_hbm):
        def idx_start(k, p):
            e0 = (s * CPT + k) * CH
            pltpu.async_copy(srcf_hbm.at[pl.ds(e0, CH)], src_v.at[p], isem[p])
            pltpu.async_copy(dstf_hbm.at[pl.ds(e0, CH)], dst_v.at[p], isem[p])

        def idx_wait(k, p):
            e0 = (s * CPT + k) * CH
            pltpu.make_async_copy(srcf_hbm.at[pl.ds(e0, CH)], src_v.at[p],
                                  isem[p]).wait()
            pltpu.make_async_copy(dstf_hbm.at[pl.ds(e0, CH)], dst_v.at[p],
                                  isem[p]).wait()

        def gather_start(p):
            pltpu.async_copy(x_hbm.at[src_v.at[p]], rows[p], gsem[p])

        def gather_wait(p):
            pltpu.make_async_copy(x_hbm.at[src_v.at[p]], rows[p],
                                  gsem[p]).wait()

        def scatter(p):
            pltpu.sync_copy(rows[p], acc_sh.at[dst_v.at[p]], add=True)

        def process(k, p0, p2, steady):
            gather_wait(p0)
            if steady:
                @pl.when(k + 2 < CPT)
                def _():
                    idx_wait(k + 2, p2)
                    gather_start(p2)
            scatter(p0)
            if steady:
                @pl.when(k + 3 < CPT)
                def _():
                    idx_start(k + 3, p0)

        idx_start(0, 0)
        idx_start(1, 1)
        idx_start(2, 2)
        idx_wait(0, 0)
        gather_start(0)
        idx_wait(1, 1)
        gather_start(1)

        def triple(j, u):
            k = 3 * j
            process(k, 0, 2, True)
            process(k + 1, 1, 0, True)
            process(k + 2, 2, 1, True)
            return u

        lax.fori_loop(0, CPT // 3, triple, 0)
        for k in range(3 * (CPT // 3), CPT):
            process(k, k % 3, (k + 2) % 3, False)

    @pl.when(c == 0)
    def _():
        run(xlo_hbm)

    @pl.when(c == 1)
    def _():
        run(xhi_hbm)

    plsc.subcore_barrier()

    @pl.when(c == 0)
    def _():
        pltpu.sync_copy(acc_sh.at[pl.ds(s * RTA, RTA)], h0_hbm.at[pl.ds(s * RTA, RTA)])

    @pl.when(c == 1)
    def _():
        pltpu.sync_copy(acc_sh.at[pl.ds(s * RTA, RTA)], h1_hbm.at[pl.ds(s * RTA, RTA)])


def _dense_body(prev_ref, raw_ref, w_ref, din_ref, dout_ref, xlo_ref, xhi_ref):
    n_in = lax.rsqrt(jnp.maximum(din_ref[...], 1.0))
    n_out = lax.rsqrt(jnp.maximum(dout_ref[...], 1.0))
    p = jnp.dot(prev_ref[...], w_ref[...], preferred_element_type=jnp.float32)
    x = raw_ref[...] * n_in + p * n_out
    xlo_ref[...] = x[:, :HD]
    xhi_ref[...] = x[:, HD:]


def _dense(prev, raw, W, din2, dout2):
    return pl.pallas_call(
        _dense_body,
        grid=(N2 // BR,),
        in_specs=[
            pl.BlockSpec((BR, D), lambda i: (i, 0)),
            pl.BlockSpec((BR, D), lambda i: (i, 0)),
            pl.BlockSpec((D, D), lambda i: (0, 0)),
            pl.BlockSpec((BR, 1), lambda i: (i, 0)),
            pl.BlockSpec((BR, 1), lambda i: (i, 0)),
        ],
        out_specs=(
            pl.BlockSpec((BR, HD), lambda i: (i, 0)),
            pl.BlockSpec((BR, HD), lambda i: (i, 0)),
        ),
        out_shape=(
            jax.ShapeDtypeStruct((N2, HD), jnp.float32),
            jax.ShapeDtypeStruct((N2, HD), jnp.float32),
        ),
    )(prev, raw, W, din2, dout2)


def _final_body(h0_ref, h1_ref, din_ref, b_ref, o_ref):
    n_in = lax.rsqrt(jnp.maximum(din_ref[...], 1.0))
    h = jnp.concatenate([h0_ref[...], h1_ref[...]], axis=1)
    o_ref[...] = jnp.maximum(h * n_in + b_ref[...], 0.0)


def _final(h0, h1, din2, b2):
    return pl.pallas_call(
        _final_body,
        grid=(NA // BRF,),
        in_specs=[
            pl.BlockSpec((BRF, HD), lambda i: (i, 0)),
            pl.BlockSpec((BRF, HD), lambda i: (i, 0)),
            pl.BlockSpec((BRF, 1), lambda i: (i, 0)),
            pl.BlockSpec((1, D), lambda i: (0, 0)),
        ],
        out_specs=pl.BlockSpec((BRF, D), lambda i: (i, 0)),
        out_shape=jax.ShapeDtypeStruct((N, D), jnp.float32),
    )(h0, h1, din2, b2)


def kernel(prev, raw, edge_index, W, b):
    npad = E2 - E
    pad_rows = N + (jnp.arange(npad, dtype=jnp.int32) % (NA - N))
    srcf = jnp.concatenate([edge_index[0], pad_rows])
    dstf = jnp.concatenate([edge_index[1], pad_rows])
    src2d = srcf.reshape(ER, CH)
    dst2d = dstf.reshape(ER, CH)
    zvec = jnp.zeros((RT,), jnp.float32)
    zrows = jnp.zeros((RTA, HD), jnp.float32)

    deg_in, deg_out = _degrees(dst2d, src2d, zvec)
    din2 = deg_in.reshape(N2, 1)
    dout2 = deg_out.reshape(N2, 1)

    xlo, xhi = _dense(prev, raw, W, din2, dout2)
    h0, h1 = _aggregate(xlo, xhi, srcf, dstf, zrows)
    return _final(h0, h1, din2, b.reshape(1, D))


# speedup vs baseline: 11.8679x; 1.0563x over previous
import functools

import jax
import jax.numpy as jnp
from jax import lax
from jax.experimental import pallas as pl
from jax.experimental.pallas import tpu as pltpu
from jax.experimental.pallas import tpu_sc as plsc

N = 10000
E = 160000
D = 256
HD = D // 2
N2 = 10240
RT = N2 // 16
NA = 10112
RTA = NA // 16
CH = 128
CPT = 80
E2 = 16 * CPT * CH
EPT = CPT * CH
ER = E2 // CH
BR = N2 // 8
BRF = NA // 8

_mesh = plsc.VectorSubcoreMesh(core_axis_name="c", subcore_axis_name="s")


@functools.partial(
    pl.kernel,
    out_type=(
        jax.ShapeDtypeStruct((N2,), jnp.float32),
        jax.ShapeDtypeStruct((N2,), jnp.float32),
    ),
    mesh=_mesh,
    scratch_types=[
        pltpu.VMEM((CPT, CH), jnp.int32),
        pltpu.VMEM((CH,), jnp.float32),
        pltpu.VMEM_SHARED((N2,), jnp.float32),
    ],
)
def _degrees(dst_hbm, src_hbm, zvec_hbm, din_hbm, dout_hbm, idx_all, ones_v, acc_sh):
    c = lax.axis_index("c")
    s = lax.axis_index("s")
    for i in range(CH // 16):
        ones_v[pl.ds(i * 16, 16)] = jnp.full((16,), 1.0, jnp.float32)
    pltpu.sync_copy(zvec_hbm, acc_sh.at[pl.ds(s * RT, RT)])

    @pl.when(c == 0)
    def _():
        pltpu.sync_copy(dst_hbm.at[pl.ds(s * CPT, CPT)], idx_all)

    @pl.when(c == 1)
    def _():
        pltpu.sync_copy(src_hbm.at[pl.ds(s * CPT, CPT)], idx_all)

    plsc.subcore_barrier()

    def chunk(k, u):
        pltpu.sync_copy(ones_v, acc_sh.at[idx_all.at[k]], add=True)
        return u

    lax.fori_loop(0, CPT, chunk, 0)
    plsc.subcore_barrier()

    @pl.when(c == 0)
    def _():
        pltpu.sync_copy(acc_sh.at[pl.ds(s * RT, RT)], din_hbm.at[pl.ds(s * RT, RT)])

    @pl.when(c == 1)
    def _():
        pltpu.sync_copy(acc_sh.at[pl.ds(s * RT, RT)], dout_hbm.at[pl.ds(s * RT, RT)])


@functools.partial(
    pl.kernel,
    out_type=(
        jax.ShapeDtypeStruct((NA, HD), jnp.float32),
        jax.ShapeDtypeStruct((NA, HD), jnp.float32),
    ),
    mesh=_mesh,
    scratch_types=[
        pltpu.VMEM((3, CH), jnp.int32),
        pltpu.VMEM((3, CH), jnp.int32),
        pltpu.VMEM((CH, HD), jnp.float32),
        pltpu.VMEM((CH, HD), jnp.float32),
        pltpu.VMEM((CH, HD), jnp.float32),
        pltpu.VMEM_SHARED((NA, HD), jnp.float32),
        pltpu.SemaphoreType.DMA,
        pltpu.SemaphoreType.DMA,
        pltpu.SemaphoreType.DMA,
        pltpu.SemaphoreType.DMA,
        pltpu.SemaphoreType.DMA,
        pltpu.SemaphoreType.DMA,
    ],
)
def _aggregate(xlo_hbm, xhi_hbm, srcf_hbm, dstf_hbm, zrows_hbm, h0_hbm, h1_hbm,
               src_v, dst_v, rows_0, rows_1, rows_2, acc_sh,
               gsem_0, gsem_1, gsem_2, isem_0, isem_1, isem_2):
    c = lax.axis_index("c")
    s = lax.axis_index("s")
    pltpu.sync_copy(zrows_hbm, acc_sh.at[pl.ds(s * RTA, RTA)])
    plsc.subcore_barrier()
    rows = (rows_0, rows_1, rows_2)
    gsem = (gsem_0, gsem_1, gsem_2)
    isem = (isem_0, isem_1, isem_2)

    def run(x_hbm):
        def idx_start(k, p):
            e0 = (s * CPT + k) * CH
            pltpu.async_copy(srcf_hbm.at[pl.ds(e0, CH)], src_v.at[p], isem[p])
            pltpu.async_copy(dstf_hbm.at[pl.ds(e0, CH)], dst_v.at[p], isem[p])

        def idx_wait(k, p):
            e0 = (s * CPT + k) * CH
            pltpu.make_async_copy(srcf_hbm.at[pl.ds(e0, CH)], src_v.at[p],
                                  isem[p]).wait()
            pltpu.make_async_copy(dstf_hbm.at[pl.ds(e0, CH)], dst_v.at[p],
                                  isem[p]).wait()

        def gather_start(p):
            pltpu.async_copy(x_hbm.at[src_v.at[p]], rows[p], gsem[p])

        def gather_wait(p):
            pltpu.make_async_copy(x_hbm.at[src_v.at[p]], rows[p],
                                  gsem[p]).wait()

        def scatter(p):
            pltpu.sync_copy(rows[p], acc_sh.at[dst_v.at[p]], add=True)

        def process(k, p0, p2, steady):
            gather_wait(p0)
            if steady:
                @pl.when(k + 2 < CPT)
                def _():
                    idx_wait(k + 2, p2)
                    gather_start(p2)
            scatter(p0)
            if steady:
                @pl.when(k + 3 < CPT)
                def _():
                    idx_start(k + 3, p0)

        idx_start(0, 0)
        idx_start(1, 1)
        idx_start(2, 2)
        idx_wait(0, 0)
        gather_start(0)
        idx_wait(1, 1)
        gather_start(1)

        def triple(j, u):
            k = 3 * j
            process(k, 0, 2, True)
            process(k + 1, 1, 0, True)
            process(k + 2, 2, 1, True)
            return u

        lax.fori_loop(0, CPT // 3, triple, 0)
        for k in range(3 * (CPT // 3), CPT):
            process(k, k % 3, (k + 2) % 3, False)

    @pl.when(c == 0)
    def _():
        run(xlo_hbm)

    @pl.when(c == 1)
    def _():
        run(xhi_hbm)

    plsc.subcore_barrier()

    @pl.when(c == 0)
    def _():
        pltpu.sync_copy(acc_sh.at[pl.ds(s * RTA, RTA)], h0_hbm.at[pl.ds(s * RTA, RTA)])

    @pl.when(c == 1)
    def _():
        pltpu.sync_copy(acc_sh.at[pl.ds(s * RTA, RTA)], h1_hbm.at[pl.ds(s * RTA, RTA)])


def _dense_body(prev_ref, raw_ref, w_ref, din_ref, dout_ref, xlo_ref, xhi_ref):
    i = pl.program_id(0)
    n_in = lax.rsqrt(jnp.maximum(din_ref[pl.ds(i * BR, BR)], 1.0))[:, None]
    n_out = lax.rsqrt(jnp.maximum(dout_ref[pl.ds(i * BR, BR)], 1.0))[:, None]
    p = jnp.dot(prev_ref[...], w_ref[...], preferred_element_type=jnp.float32)
    x = raw_ref[...] * n_in + p * n_out
    xlo_ref[...] = x[:, :HD]
    xhi_ref[...] = x[:, HD:]


def _dense(prev, raw, W, din2, dout2):
    return pl.pallas_call(
        _dense_body,
        grid=(N2 // BR,),
        in_specs=[
            pl.BlockSpec((BR, D), lambda i: (i, 0)),
            pl.BlockSpec((BR, D), lambda i: (i, 0)),
            pl.BlockSpec((D, D), lambda i: (0, 0)),
            pl.BlockSpec((N2,), lambda i: (0,)),
            pl.BlockSpec((N2,), lambda i: (0,)),
        ],
        out_specs=(
            pl.BlockSpec((BR, HD), lambda i: (i, 0)),
            pl.BlockSpec((BR, HD), lambda i: (i, 0)),
        ),
        out_shape=(
            jax.ShapeDtypeStruct((N2, HD), jnp.float32),
            jax.ShapeDtypeStruct((N2, HD), jnp.float32),
        ),
    )(prev, raw, W, din2, dout2)


def _final_body(h0_ref, h1_ref, din_ref, b_ref, o_ref):
    i = pl.program_id(0)
    n_in = lax.rsqrt(jnp.maximum(din_ref[pl.ds(i * BR, BR)], 1.0))[:, None]
    h = jnp.concatenate([h0_ref[...], h1_ref[...]], axis=1)
    o_ref[...] = jnp.maximum(h * n_in + b_ref[...], 0.0)


def _final(h0, h1, din2, b2):
    return pl.pallas_call(
        _final_body,
        grid=(N2 // BR,),
        in_specs=[
            pl.BlockSpec((BR, HD), lambda i: (i, 0)),
            pl.BlockSpec((BR, HD), lambda i: (i, 0)),
            pl.BlockSpec((N2,), lambda i: (0,)),
            pl.BlockSpec((1, D), lambda i: (0, 0)),
        ],
        out_specs=pl.BlockSpec((BR, D), lambda i: (i, 0)),
        out_shape=jax.ShapeDtypeStruct((N, D), jnp.float32),
    )(h0, h1, din2, b2)


def kernel(prev, raw, edge_index, W, b):
    npad = E2 - E
    pad_rows = N + (jnp.arange(npad, dtype=jnp.int32) % (NA - N))
    srcf = jnp.concatenate([edge_index[0], pad_rows])
    dstf = jnp.concatenate([edge_index[1], pad_rows])
    src2d = srcf.reshape(ER, CH)
    dst2d = dstf.reshape(ER, CH)
    zvec = jnp.zeros((RT,), jnp.float32)
    zrows = jnp.zeros((RTA, HD), jnp.float32)

    deg_in, deg_out = _degrees(dst2d, src2d, zvec)

    xlo, xhi = _dense(prev, raw, W, deg_in, deg_out)
    h0, h1 = _aggregate(xlo, xhi, srcf, dstf, zrows)
    return _final(h0, h1, deg_in, b.reshape(1, D))


# speedup vs baseline: 13.6148x; 1.1472x over previous
import functools

import jax
import jax.numpy as jnp
from jax import lax
from jax.experimental import pallas as pl
from jax.experimental.pallas import tpu as pltpu
from jax.experimental.pallas import tpu_sc as plsc

N = 10000
E = 160000
D = 256
HD = D // 2
N2 = 10240
RT = N2 // 16
NA = 10112
RTA = NA // 16
CH = 128
CPT = 80
E2 = 16 * CPT * CH
EPT = CPT * CH
ER = E2 // CH
BR = N2 // 8
BRF = NA // 8

_mesh = plsc.VectorSubcoreMesh(core_axis_name="c", subcore_axis_name="s")


@functools.partial(
    pl.kernel,
    out_type=(
        jax.ShapeDtypeStruct((N2,), jnp.float32),
        jax.ShapeDtypeStruct((N2,), jnp.float32),
    ),
    mesh=_mesh,
    scratch_types=[
        pltpu.VMEM((CPT, CH), jnp.int32),
        pltpu.VMEM((CH,), jnp.float32),
        pltpu.VMEM_SHARED((N2,), jnp.float32),
        pltpu.SemaphoreType.DMA,
    ],
)
def _degrees(dst_hbm, src_hbm, zvec_hbm, din_hbm, dout_hbm, idx_all, ones_v, acc_sh,
             dsem):
    c = lax.axis_index("c")
    s = lax.axis_index("s")
    for i in range(CH // 16):
        ones_v[pl.ds(i * 16, 16)] = jnp.full((16,), 1.0, jnp.float32)
    pltpu.sync_copy(zvec_hbm, acc_sh.at[pl.ds(s * RT, RT)])

    @pl.when(c == 0)
    def _():
        pltpu.sync_copy(dst_hbm.at[pl.ds(s * CPT, CPT)], idx_all)

    @pl.when(c == 1)
    def _():
        pltpu.sync_copy(src_hbm.at[pl.ds(s * CPT, CPT)], idx_all)

    plsc.subcore_barrier()

    def chunk(k, u):
        pltpu.async_copy(ones_v, acc_sh.at[idx_all.at[k]], dsem, add=True)
        return u

    def drain(k, u):
        pltpu.make_async_copy(ones_v, acc_sh.at[idx_all.at[k]], dsem).wait()
        return u

    lax.fori_loop(0, CPT, chunk, 0)
    lax.fori_loop(0, CPT, drain, 0)
    plsc.subcore_barrier()

    @pl.when(c == 0)
    def _():
        pltpu.sync_copy(acc_sh.at[pl.ds(s * RT, RT)], din_hbm.at[pl.ds(s * RT, RT)])

    @pl.when(c == 1)
    def _():
        pltpu.sync_copy(acc_sh.at[pl.ds(s * RT, RT)], dout_hbm.at[pl.ds(s * RT, RT)])


@functools.partial(
    pl.kernel,
    out_type=(
        jax.ShapeDtypeStruct((NA, HD), jnp.float32),
        jax.ShapeDtypeStruct((NA, HD), jnp.float32),
    ),
    mesh=_mesh,
    scratch_types=[
        pltpu.VMEM((3, CH), jnp.int32),
        pltpu.VMEM((4, CH), jnp.int32),
        pltpu.VMEM((CH, HD), jnp.float32),
        pltpu.VMEM((CH, HD), jnp.float32),
        pltpu.VMEM((CH, HD), jnp.float32),
        pltpu.VMEM_SHARED((NA, HD), jnp.float32),
        pltpu.SemaphoreType.DMA,
        pltpu.SemaphoreType.DMA,
        pltpu.SemaphoreType.DMA,
        pltpu.SemaphoreType.DMA,
        pltpu.SemaphoreType.DMA,
        pltpu.SemaphoreType.DMA,
    ],
)
def _aggregate(xlo_hbm, xhi_hbm, srcf_hbm, dstf_hbm, zrows_hbm, h0_hbm, h1_hbm,
               src_v, dst_v, rows_0, rows_1, rows_2, acc_sh,
               gsem_0, gsem_1, gsem_2, isem_0, isem_1, isem_2):
    c = lax.axis_index("c")
    s = lax.axis_index("s")
    pltpu.sync_copy(zrows_hbm, acc_sh.at[pl.ds(s * RTA, RTA)])
    plsc.subcore_barrier()
    rows = (rows_0, rows_1, rows_2)
    gsem = (gsem_0, gsem_1, gsem_2)
    isem = (isem_0, isem_1, isem_2)

    def run(x_hbm):
        def idx_start(k, ps, qd):
            e0 = (s * CPT + k) * CH
            pltpu.async_copy(srcf_hbm.at[pl.ds(e0, CH)], src_v.at[ps], isem[ps])
            pltpu.async_copy(dstf_hbm.at[pl.ds(e0, CH)], dst_v.at[qd], isem[ps])

        def idx_wait(k, ps, qd):
            e0 = (s * CPT + k) * CH
            pltpu.make_async_copy(srcf_hbm.at[pl.ds(e0, CH)], src_v.at[ps],
                                  isem[ps]).wait()
            pltpu.make_async_copy(dstf_hbm.at[pl.ds(e0, CH)], dst_v.at[qd],
                                  isem[ps]).wait()

        def gather_start(p):
            pltpu.async_copy(x_hbm.at[src_v.at[p]], rows[p], gsem[p])

        def gather_wait(p):
            pltpu.make_async_copy(x_hbm.at[src_v.at[p]], rows[p],
                                  gsem[p]).wait()

        def scatter_start(p, q):
            pltpu.async_copy(rows[p], acc_sh.at[dst_v.at[q]], gsem[p],
                             add=True)

        def scatter_wait(p, q):
            pltpu.make_async_copy(rows[p], acc_sh.at[dst_v.at[q]],
                                  gsem[p]).wait()

        def process(k, m, wait_prev, prefetch, start_next):
            pr, qd = m % 3, m % 4
            pm3, pm4 = (m - 1) % 3, (m - 1) % 4
            gather_wait(pr)
            scatter_start(pr, qd)
            if wait_prev == "traced":
                @pl.when(k > 0)
                def _():
                    scatter_wait(pm3, pm4)
            elif wait_prev:
                scatter_wait(pm3, pm4)
            if prefetch:
                idx_wait(k + 2, (m + 2) % 3, (m + 2) % 4)
                gather_start((m + 2) % 3)
            if start_next:
                idx_start(k + 3, (m + 3) % 3, (m + 3) % 4)

        idx_start(0, 0, 0)
        idx_start(1, 1, 1)
        idx_start(2, 2, 2)
        idx_wait(0, 0, 0)
        gather_start(0)
        idx_wait(1, 1, 1)
        gather_start(1)

        def twelve(j, u):
            k0 = 12 * j
            for m in range(12):
                process(k0 + m, m, "traced" if m == 0 else True, True, True)
            return u

        lax.fori_loop(0, CPT // 12, twelve, 0)
        for k in range(12 * (CPT // 12), CPT):
            m = k % 12
            process(k, m, True, k + 2 < CPT, k + 3 < CPT)
        scatter_wait((CPT - 1) % 3, (CPT - 1) % 4)

    @pl.when(c == 0)
    def _():
        run(xlo_hbm)

    @pl.when(c == 1)
    def _():
        run(xhi_hbm)

    plsc.subcore_barrier()

    @pl.when(c == 0)
    def _():
        pltpu.sync_copy(acc_sh.at[pl.ds(s * RTA, RTA)], h0_hbm.at[pl.ds(s * RTA, RTA)])

    @pl.when(c == 1)
    def _():
        pltpu.sync_copy(acc_sh.at[pl.ds(s * RTA, RTA)], h1_hbm.at[pl.ds(s * RTA, RTA)])


def _dense_body(prev_ref, raw_ref, w_ref, din_ref, dout_ref, xlo_ref, xhi_ref):
    i = pl.program_id(0)
    n_in = lax.rsqrt(jnp.maximum(din_ref[pl.ds(i * BR, BR)], 1.0))[:, None]
    n_out = lax.rsqrt(jnp.maximum(dout_ref[pl.ds(i * BR, BR)], 1.0))[:, None]
    p = jnp.dot(prev_ref[...], w_ref[...], preferred_element_type=jnp.float32)
    x = raw_ref[...] * n_in + p * n_out
    xlo_ref[...] = x[:, :HD]
    xhi_ref[...] = x[:, HD:]


def _dense(prev, raw, W, din2, dout2):
    return pl.pallas_call(
        _dense_body,
        grid=(N2 // BR,),
        in_specs=[
            pl.BlockSpec((BR, D), lambda i: (i, 0)),
            pl.BlockSpec((BR, D), lambda i: (i, 0)),
            pl.BlockSpec((D, D), lambda i: (0, 0)),
            pl.BlockSpec((N2,), lambda i: (0,)),
            pl.BlockSpec((N2,), lambda i: (0,)),
        ],
        out_specs=(
            pl.BlockSpec((BR, HD), lambda i: (i, 0)),
            pl.BlockSpec((BR, HD), lambda i: (i, 0)),
        ),
        out_shape=(
            jax.ShapeDtypeStruct((N2, HD), jnp.float32),
            jax.ShapeDtypeStruct((N2, HD), jnp.float32),
        ),
    )(prev, raw, W, din2, dout2)


def _final_body(h0_ref, h1_ref, din_ref, b_ref, o_ref):
    i = pl.program_id(0)
    n_in = lax.rsqrt(jnp.maximum(din_ref[pl.ds(i * BR, BR)], 1.0))[:, None]
    h = jnp.concatenate([h0_ref[...], h1_ref[...]], axis=1)
    o_ref[...] = jnp.maximum(h * n_in + b_ref[...], 0.0)


def _final(h0, h1, din2, b2):
    return pl.pallas_call(
        _final_body,
        grid=(N2 // BR,),
        in_specs=[
            pl.BlockSpec((BR, HD), lambda i: (i, 0)),
            pl.BlockSpec((BR, HD), lambda i: (i, 0)),
            pl.BlockSpec((N2,), lambda i: (0,)),
            pl.BlockSpec((1, D), lambda i: (0, 0)),
        ],
        out_specs=pl.BlockSpec((BR, D), lambda i: (i, 0)),
        out_shape=jax.ShapeDtypeStruct((N, D), jnp.float32),
    )(h0, h1, din2, b2)


def kernel(prev, raw, edge_index, W, b):
    npad = E2 - E
    pad_rows = N + (jnp.arange(npad, dtype=jnp.int32) % (NA - N))
    srcf = jnp.concatenate([edge_index[0], pad_rows])
    dstf = jnp.concatenate([edge_index[1], pad_rows])
    src2d = srcf.reshape(ER, CH)
    dst2d = dstf.reshape(ER, CH)
    zvec = jnp.zeros((RT,), jnp.float32)
    zrows = jnp.zeros((RTA, HD), jnp.float32)

    deg_in, deg_out = _degrees(dst2d, src2d, zvec)

    xlo, xhi = _dense(prev, raw, W, deg_in, deg_out)
    h0, h1 = _aggregate(xlo, xhi, srcf, dstf, zrows)
    return _final(h0, h1, deg_in, b.reshape(1, D))


# speedup vs baseline: 14.1219x; 1.0372x over previous
import functools

import jax
import jax.numpy as jnp
from jax import lax
from jax.experimental import pallas as pl
from jax.experimental.pallas import tpu as pltpu
from jax.experimental.pallas import tpu_sc as plsc

N = 10000
E = 160000
D = 256
HD = D // 2
N2 = 10240
RT = N2 // 16
NA = 10112
RTA = NA // 16
CH = 128
CPT = 80
E2 = 16 * CPT * CH
EPT = CPT * CH
ER = E2 // CH
BR = N2 // 8
BRF = NA // 8

_mesh = plsc.VectorSubcoreMesh(core_axis_name="c", subcore_axis_name="s")


@functools.partial(
    pl.kernel,
    out_type=(
        jax.ShapeDtypeStruct((N2,), jnp.float32),
        jax.ShapeDtypeStruct((N2,), jnp.float32),
    ),
    mesh=_mesh,
    scratch_types=[
        pltpu.VMEM((CPT, CH), jnp.int32),
        pltpu.VMEM((CH,), jnp.float32),
        pltpu.VMEM_SHARED((N2,), jnp.float32),
        pltpu.SemaphoreType.DMA,
    ],
)
def _degrees(dst_hbm, src_hbm, zvec_hbm, din_hbm, dout_hbm, idx_all, ones_v, acc_sh,
             dsem):
    c = lax.axis_index("c")
    s = lax.axis_index("s")
    for i in range(CH // 16):
        ones_v[pl.ds(i * 16, 16)] = jnp.full((16,), 1.0, jnp.float32)
    pltpu.sync_copy(zvec_hbm, acc_sh.at[pl.ds(s * RT, RT)])

    @pl.when(c == 0)
    def _():
        pltpu.sync_copy(dst_hbm.at[pl.ds(s * CPT, CPT)], idx_all)

    @pl.when(c == 1)
    def _():
        pltpu.sync_copy(src_hbm.at[pl.ds(s * CPT, CPT)], idx_all)

    plsc.subcore_barrier()

    def chunk(k, u):
        pltpu.async_copy(ones_v, acc_sh.at[idx_all.at[k]], dsem, add=True)
        return u

    def drain(k, u):
        pltpu.make_async_copy(ones_v, acc_sh.at[idx_all.at[k]], dsem).wait()
        return u

    lax.fori_loop(0, CPT, chunk, 0)
    lax.fori_loop(0, CPT, drain, 0)
    plsc.subcore_barrier()

    @pl.when(c == 0)
    def _():
        pltpu.sync_copy(acc_sh.at[pl.ds(s * RT, RT)], din_hbm.at[pl.ds(s * RT, RT)])

    @pl.when(c == 1)
    def _():
        pltpu.sync_copy(acc_sh.at[pl.ds(s * RT, RT)], dout_hbm.at[pl.ds(s * RT, RT)])


@functools.partial(
    pl.kernel,
    out_type=(
        jax.ShapeDtypeStruct((NA, HD), jnp.float32),
        jax.ShapeDtypeStruct((NA, HD), jnp.float32),
    ),
    mesh=_mesh,
    scratch_types=[
        pltpu.VMEM((3, CH), jnp.int32),
        pltpu.VMEM((4, CH), jnp.int32),
        pltpu.VMEM((CH, HD), jnp.float32),
        pltpu.VMEM((CH, HD), jnp.float32),
        pltpu.VMEM((CH, HD), jnp.float32),
        pltpu.VMEM_SHARED((NA, HD), jnp.float32),
        pltpu.SemaphoreType.DMA,
        pltpu.SemaphoreType.DMA,
        pltpu.SemaphoreType.DMA,
        pltpu.SemaphoreType.DMA,
        pltpu.SemaphoreType.DMA,
        pltpu.SemaphoreType.DMA,
    ],
)
def _aggregate(xlo_hbm, xhi_hbm, srcf_hbm, dstf_hbm, zrows_hbm, h0_hbm, h1_hbm,
               src_v, dst_v, rows_0, rows_1, rows_2, acc_sh,
               gsem_0, gsem_1, gsem_2, isem_0, isem_1, isem_2):
    c = lax.axis_index("c")
    s = lax.axis_index("s")
    pltpu.sync_copy(zrows_hbm, acc_sh.at[pl.ds(s * RTA, RTA)])
    plsc.subcore_barrier()
    rows = (rows_0, rows_1, rows_2)
    gsem = (gsem_0, gsem_1, gsem_2)
    isem = (isem_0, isem_1, isem_2)

    def run(x_hbm):
        def idx_start(k, ps, qd):
            e0 = (s * CPT + k) * CH
            pltpu.async_copy(srcf_hbm.at[pl.ds(e0, CH)], src_v.at[ps], isem[ps])
            pltpu.async_copy(dstf_hbm.at[pl.ds(e0, CH)], dst_v.at[qd], isem[ps])

        def idx_wait(k, ps, qd):
            e0 = (s * CPT + k) * CH
            pltpu.make_async_copy(srcf_hbm.at[pl.ds(e0, CH)], src_v.at[ps],
                                  isem[ps]).wait()
            pltpu.make_async_copy(dstf_hbm.at[pl.ds(e0, CH)], dst_v.at[qd],
                                  isem[ps]).wait()

        def gather_start(p):
            pltpu.async_copy(x_hbm.at[src_v.at[p]], rows[p], gsem[p])

        def gather_wait(p):
            pltpu.make_async_copy(x_hbm.at[src_v.at[p]], rows[p],
                                  gsem[p]).wait()

        def scatter_start(p, q):
            pltpu.async_copy(rows[p], acc_sh.at[dst_v.at[q]], gsem[p],
                             add=True)

        def scatter_wait(p, q):
            pltpu.make_async_copy(rows[p], acc_sh.at[dst_v.at[q]],
                                  gsem[p]).wait()

        def process(k, m, wait_prev, prefetch, start_next):
            pr, qd = m % 3, m % 4
            pm3, pm4 = (m - 1) % 3, (m - 1) % 4
            gather_wait(pr)
            scatter_start(pr, qd)
            if wait_prev == "traced":
                @pl.when(k > 0)
                def _():
                    scatter_wait(pm3, pm4)
            elif wait_prev:
                scatter_wait(pm3, pm4)
            if prefetch:
                idx_wait(k + 2, (m + 2) % 3, (m + 2) % 4)
                gather_start((m + 2) % 3)
            if start_next:
                idx_start(k + 3, (m + 3) % 3, (m + 3) % 4)

        idx_start(0, 0, 0)
        idx_start(1, 1, 1)
        idx_start(2, 2, 2)
        idx_wait(0, 0, 0)
        gather_start(0)
        idx_wait(1, 1, 1)
        gather_start(1)

        def twelve(j, u):
            k0 = 12 * j
            for m in range(12):
                process(k0 + m, m, "traced" if m == 0 else True, True, True)
            return u

        lax.fori_loop(0, CPT // 12, twelve, 0)
        for k in range(12 * (CPT // 12), CPT):
            m = k % 12
            process(k, m, True, k + 2 < CPT, k + 3 < CPT)
        scatter_wait((CPT - 1) % 3, (CPT - 1) % 4)

    @pl.when(c == 0)
    def _():
        run(xlo_hbm)

    @pl.when(c == 1)
    def _():
        run(xhi_hbm)

    plsc.subcore_barrier()

    @pl.when(c == 0)
    def _():
        pltpu.sync_copy(acc_sh.at[pl.ds(s * RTA, RTA)], h0_hbm.at[pl.ds(s * RTA, RTA)])

    @pl.when(c == 1)
    def _():
        pltpu.sync_copy(acc_sh.at[pl.ds(s * RTA, RTA)], h1_hbm.at[pl.ds(s * RTA, RTA)])


def _edgeprep_body(ei_ref, srcf_ref, dstf_ref):
    pad = N + (lax.broadcasted_iota(jnp.int32, (E2 - E,), 0) % (NA - N))
    srcf_ref[...] = jnp.concatenate([ei_ref[0, :], pad])
    dstf_ref[...] = jnp.concatenate([ei_ref[1, :], pad])


def _edgeprep(edge_index):
    return pl.pallas_call(
        _edgeprep_body,
        grid=(1,),
        in_specs=[pl.BlockSpec((2, E), lambda i: (0, 0))],
        out_specs=(
            pl.BlockSpec((E2,), lambda i: (0,)),
            pl.BlockSpec((E2,), lambda i: (0,)),
        ),
        out_shape=(
            jax.ShapeDtypeStruct((E2,), jnp.int32),
            jax.ShapeDtypeStruct((E2,), jnp.int32),
        ),
    )(edge_index)


def _dense_body(prev_ref, raw_ref, w_ref, din_ref, dout_ref, xlo_ref, xhi_ref):
    i = pl.program_id(0)
    n_in = lax.rsqrt(jnp.maximum(din_ref[pl.ds(i * BR, BR)], 1.0))[:, None]
    n_out = lax.rsqrt(jnp.maximum(dout_ref[pl.ds(i * BR, BR)], 1.0))[:, None]
    p = jnp.dot(prev_ref[...], w_ref[...], preferred_element_type=jnp.float32)
    x = raw_ref[...] * n_in + p * n_out
    xlo_ref[...] = x[:, :HD]
    xhi_ref[...] = x[:, HD:]


def _dense(prev, raw, W, din2, dout2):
    return pl.pallas_call(
        _dense_body,
        grid=(N2 // BR,),
        in_specs=[
            pl.BlockSpec((BR, D), lambda i: (i, 0)),
            pl.BlockSpec((BR, D), lambda i: (i, 0)),
            pl.BlockSpec((D, D), lambda i: (0, 0)),
            pl.BlockSpec((N2,), lambda i: (0,)),
            pl.BlockSpec((N2,), lambda i: (0,)),
        ],
        out_specs=(
            pl.BlockSpec((BR, HD), lambda i: (i, 0)),
            pl.BlockSpec((BR, HD), lambda i: (i, 0)),
        ),
        out_shape=(
            jax.ShapeDtypeStruct((N2, HD), jnp.float32),
            jax.ShapeDtypeStruct((N2, HD), jnp.float32),
        ),
    )(prev, raw, W, din2, dout2)


def _final_body(h0_ref, h1_ref, din_ref, b_ref, o_ref):
    i = pl.program_id(0)
    n_in = lax.rsqrt(jnp.maximum(din_ref[pl.ds(i * BR, BR)], 1.0))[:, None]
    h = jnp.concatenate([h0_ref[...], h1_ref[...]], axis=1)
    o_ref[...] = jnp.maximum(h * n_in + b_ref[...], 0.0)


def _final(h0, h1, din2, b2):
    return pl.pallas_call(
        _final_body,
        grid=(N2 // BR,),
        in_specs=[
            pl.BlockSpec((BR, HD), lambda i: (i, 0)),
            pl.BlockSpec((BR, HD), lambda i: (i, 0)),
            pl.BlockSpec((N2,), lambda i: (0,)),
            pl.BlockSpec((1, D), lambda i: (0, 0)),
        ],
        out_specs=pl.BlockSpec((BR, D), lambda i: (i, 0)),
        out_shape=jax.ShapeDtypeStruct((N, D), jnp.float32),
    )(h0, h1, din2, b2)


def kernel(prev, raw, edge_index, W, b):
    srcf, dstf = _edgeprep(edge_index)
    src2d = srcf.reshape(ER, CH)
    dst2d = dstf.reshape(ER, CH)
    zvec = jnp.zeros((RT,), jnp.float32)
    zrows = jnp.zeros((RTA, HD), jnp.float32)

    deg_in, deg_out = _degrees(dst2d, src2d, zvec)

    xlo, xhi = _dense(prev, raw, W, deg_in, deg_out)
    h0, h1 = _aggregate(xlo, xhi, srcf, dstf, zrows)
    return _final(h0, h1, deg_in, b.reshape(1, D))


# speedup vs baseline: 14.8486x; 1.0515x over previous
import functools

import jax
import jax.numpy as jnp
from jax import lax
from jax.experimental import pallas as pl
from jax.experimental.pallas import tpu as pltpu
from jax.experimental.pallas import tpu_sc as plsc

N = 10000
E = 160000
D = 256
HD = D // 2
N2 = 10240
RT = N2 // 16
NA = 10112
RTA = NA // 16
CH = 128
CPT = 80
E2 = 16 * CPT * CH
EPT = CPT * CH
ER = E2 // CH
BR = N2 // 8
BRF = NA // 8

_mesh = plsc.VectorSubcoreMesh(core_axis_name="c", subcore_axis_name="s")


@functools.partial(
    pl.kernel,
    out_type=(
        jax.ShapeDtypeStruct((N2,), jnp.float32),
        jax.ShapeDtypeStruct((N2,), jnp.float32),
    ),
    mesh=_mesh,
    scratch_types=[
        pltpu.VMEM((CPT, CH), jnp.int32),
        pltpu.VMEM((CH,), jnp.float32),
        pltpu.VMEM((CH,), jnp.float32),
        pltpu.VMEM_SHARED((N2,), jnp.float32),
        pltpu.SemaphoreType.DMA,
    ],
)
def _degrees(dst_hbm, src_hbm, din_hbm, dout_hbm, idx_all, ones_v, zb_v, acc_sh,
             dsem):
    c = lax.axis_index("c")
    s = lax.axis_index("s")
    for i in range(CH // 16):
        ones_v[pl.ds(i * 16, 16)] = jnp.full((16,), 1.0, jnp.float32)
        zb_v[pl.ds(i * 16, 16)] = jnp.zeros((16,), jnp.float32)
    for k in range(RT // CH):
        pltpu.sync_copy(zb_v, acc_sh.at[pl.ds(s * RT + k * CH, CH)])

    @pl.when(c == 0)
    def _():
        pltpu.sync_copy(dst_hbm.at[pl.ds(s * CPT, CPT)], idx_all)

    @pl.when(c == 1)
    def _():
        pltpu.sync_copy(src_hbm.at[pl.ds(s * CPT, CPT)], idx_all)

    plsc.subcore_barrier()

    def chunk(k, u):
        pltpu.async_copy(ones_v, acc_sh.at[idx_all.at[k]], dsem, add=True)
        return u

    def drain(k, u):
        pltpu.make_async_copy(ones_v, acc_sh.at[idx_all.at[k]], dsem).wait()
        return u

    lax.fori_loop(0, CPT, chunk, 0)
    lax.fori_loop(0, CPT, drain, 0)
    plsc.subcore_barrier()

    @pl.when(c == 0)
    def _():
        pltpu.sync_copy(acc_sh.at[pl.ds(s * RT, RT)], din_hbm.at[pl.ds(s * RT, RT)])

    @pl.when(c == 1)
    def _():
        pltpu.sync_copy(acc_sh.at[pl.ds(s * RT, RT)], dout_hbm.at[pl.ds(s * RT, RT)])


@functools.partial(
    pl.kernel,
    out_type=(
        jax.ShapeDtypeStruct((NA, HD), jnp.float32),
        jax.ShapeDtypeStruct((NA, HD), jnp.float32),
    ),
    mesh=_mesh,
    scratch_types=[
        pltpu.VMEM((3, CH), jnp.int32),
        pltpu.VMEM((4, CH), jnp.int32),
        pltpu.VMEM((CH, HD), jnp.float32),
        pltpu.VMEM((CH, HD), jnp.float32),
        pltpu.VMEM((CH, HD), jnp.float32),
        pltpu.VMEM_SHARED((NA, HD), jnp.float32),
        pltpu.SemaphoreType.DMA,
        pltpu.SemaphoreType.DMA,
        pltpu.SemaphoreType.DMA,
        pltpu.SemaphoreType.DMA,
        pltpu.SemaphoreType.DMA,
        pltpu.SemaphoreType.DMA,
    ],
)
def _aggregate(xlo_hbm, xhi_hbm, srcf_hbm, dstf_hbm, h0_hbm, h1_hbm,
               src_v, dst_v, rows_0, rows_1, rows_2, acc_sh,
               gsem_0, gsem_1, gsem_2, isem_0, isem_1, isem_2):
    c = lax.axis_index("c")
    s = lax.axis_index("s")

    def zrow(r, u):
        for j in range(HD // 16):
            rows_0[r, pl.ds(j * 16, 16)] = jnp.zeros((16,), jnp.float32)
        return u

    lax.fori_loop(0, CH, zrow, 0)
    for k in range(RTA // CH):
        pltpu.sync_copy(rows_0, acc_sh.at[pl.ds(s * RTA + k * CH, CH)])
    pltpu.sync_copy(rows_0.at[pl.ds(0, RTA - (RTA // CH) * CH)],
                    acc_sh.at[pl.ds(s * RTA + (RTA // CH) * CH,
                                    RTA - (RTA // CH) * CH)])
    plsc.subcore_barrier()
    rows = (rows_0, rows_1, rows_2)
    gsem = (gsem_0, gsem_1, gsem_2)
    isem = (isem_0, isem_1, isem_2)

    def run(x_hbm):
        def idx_start(k, ps, qd):
            e0 = (s * CPT + k) * CH
            pltpu.async_copy(srcf_hbm.at[pl.ds(e0, CH)], src_v.at[ps], isem[ps])
            pltpu.async_copy(dstf_hbm.at[pl.ds(e0, CH)], dst_v.at[qd], isem[ps])

        def idx_wait(k, ps, qd):
            e0 = (s * CPT + k) * CH
            pltpu.make_async_copy(srcf_hbm.at[pl.ds(e0, CH)], src_v.at[ps],
                                  isem[ps]).wait()
            pltpu.make_async_copy(dstf_hbm.at[pl.ds(e0, CH)], dst_v.at[qd],
                                  isem[ps]).wait()

        def gather_start(p):
            pltpu.async_copy(x_hbm.at[src_v.at[p]], rows[p], gsem[p])

        def gather_wait(p):
            pltpu.make_async_copy(x_hbm.at[src_v.at[p]], rows[p],
                                  gsem[p]).wait()

        def scatter_start(p, q):
            pltpu.async_copy(rows[p], acc_sh.at[dst_v.at[q]], gsem[p],
                             add=True)

        def scatter_wait(p, q):
            pltpu.make_async_copy(rows[p], acc_sh.at[dst_v.at[q]],
                                  gsem[p]).wait()

        def process(k, m, wait_prev, prefetch, start_next):
            pr, qd = m % 3, m % 4
            pm3, pm4 = (m - 1) % 3, (m - 1) % 4
            gather_wait(pr)
            scatter_start(pr, qd)
            if wait_prev == "traced":
                @pl.when(k > 0)
                def _():
                    scatter_wait(pm3, pm4)
            elif wait_prev:
                scatter_wait(pm3, pm4)
            if prefetch:
                idx_wait(k + 2, (m + 2) % 3, (m + 2) % 4)
                gather_start((m + 2) % 3)
            if start_next:
                idx_start(k + 3, (m + 3) % 3, (m + 3) % 4)

        idx_start(0, 0, 0)
        idx_start(1, 1, 1)
        idx_start(2, 2, 2)
        idx_wait(0, 0, 0)
        gather_start(0)
        idx_wait(1, 1, 1)
        gather_start(1)

        def twelve(j, u):
            k0 = 12 * j
            for m in range(12):
                process(k0 + m, m, "traced" if m == 0 else True, True, True)
            return u

        lax.fori_loop(0, CPT // 12, twelve, 0)
        for k in range(12 * (CPT // 12), CPT):
            m = k % 12
            process(k, m, True, k + 2 < CPT, k + 3 < CPT)
        scatter_wait((CPT - 1) % 3, (CPT - 1) % 4)

    @pl.when(c == 0)
    def _():
        run(xlo_hbm)

    @pl.when(c == 1)
    def _():
        run(xhi_hbm)

    plsc.subcore_barrier()

    @pl.when(c == 0)
    def _():
        pltpu.sync_copy(acc_sh.at[pl.ds(s * RTA, RTA)], h0_hbm.at[pl.ds(s * RTA, RTA)])

    @pl.when(c == 1)
    def _():
        pltpu.sync_copy(acc_sh.at[pl.ds(s * RTA, RTA)], h1_hbm.at[pl.ds(s * RTA, RTA)])


def _edgeprep_body(ei_ref, srcf_ref, dstf_ref):
    pad = N + (lax.broadcasted_iota(jnp.int32, (E2 - E,), 0) % (NA - N))
    srcf_ref[...] = jnp.concatenate([ei_ref[0, :], pad])
    dstf_ref[...] = jnp.concatenate([ei_ref[1, :], pad])


def _edgeprep(edge_index):
    return pl.pallas_call(
        _edgeprep_body,
        grid=(1,),
        in_specs=[pl.BlockSpec((2, E), lambda i: (0, 0))],
        out_specs=(
            pl.BlockSpec((E2,), lambda i: (0,)),
            pl.BlockSpec((E2,), lambda i: (0,)),
        ),
        out_shape=(
            jax.ShapeDtypeStruct((E2,), jnp.int32),
            jax.ShapeDtypeStruct((E2,), jnp.int32),
        ),
    )(edge_index)


def _dense_body(prev_ref, raw_ref, w_ref, din_ref, dout_ref, xlo_ref, xhi_ref):
    i = pl.program_id(0)
    n_in = lax.rsqrt(jnp.maximum(din_ref[pl.ds(i * BR, BR)], 1.0))[:, None]
    n_out = lax.rsqrt(jnp.maximum(dout_ref[pl.ds(i * BR, BR)], 1.0))[:, None]
    p = jnp.dot(prev_ref[...], w_ref[...], preferred_element_type=jnp.float32)
    x = raw_ref[...] * n_in + p * n_out
    xlo_ref[...] = x[:, :HD]
    xhi_ref[...] = x[:, HD:]


def _dense(prev, raw, W, din2, dout2):
    return pl.pallas_call(
        _dense_body,
        grid=(N2 // BR,),
        in_specs=[
            pl.BlockSpec((BR, D), lambda i: (i, 0)),
            pl.BlockSpec((BR, D), lambda i: (i, 0)),
            pl.BlockSpec((D, D), lambda i: (0, 0)),
            pl.BlockSpec((N2,), lambda i: (0,)),
            pl.BlockSpec((N2,), lambda i: (0,)),
        ],
        out_specs=(
            pl.BlockSpec((BR, HD), lambda i: (i, 0)),
            pl.BlockSpec((BR, HD), lambda i: (i, 0)),
        ),
        out_shape=(
            jax.ShapeDtypeStruct((N2, HD), jnp.float32),
            jax.ShapeDtypeStruct((N2, HD), jnp.float32),
        ),
    )(prev, raw, W, din2, dout2)


def _final_body(h0_ref, h1_ref, din_ref, b_ref, o_ref):
    i = pl.program_id(0)
    n_in = lax.rsqrt(jnp.maximum(din_ref[pl.ds(i * BR, BR)], 1.0))[:, None]
    h = jnp.concatenate([h0_ref[...], h1_ref[...]], axis=1)
    o_ref[...] = jnp.maximum(h * n_in + b_ref[...], 0.0)


def _final(h0, h1, din2, b2):
    return pl.pallas_call(
        _final_body,
        grid=(N2 // BR,),
        in_specs=[
            pl.BlockSpec((BR, HD), lambda i: (i, 0)),
            pl.BlockSpec((BR, HD), lambda i: (i, 0)),
            pl.BlockSpec((N2,), lambda i: (0,)),
            pl.BlockSpec((1, D), lambda i: (0, 0)),
        ],
        out_specs=pl.BlockSpec((BR, D), lambda i: (i, 0)),
        out_shape=jax.ShapeDtypeStruct((N, D), jnp.float32),
    )(h0, h1, din2, b2)


def kernel(prev, raw, edge_index, W, b):
    srcf, dstf = _edgeprep(edge_index)
    src2d = srcf.reshape(ER, CH)
    dst2d = dstf.reshape(ER, CH)
    deg_in, deg_out = _degrees(dst2d, src2d)

    xlo, xhi = _dense(prev, raw, W, deg_in, deg_out)
    h0, h1 = _aggregate(xlo, xhi, srcf, dstf)
    return _final(h0, h1, deg_in, b.reshape(1, D))


# speedup vs baseline: 15.0189x; 1.0115x over previous
import functools

import jax
import jax.numpy as jnp
from jax import lax
from jax.experimental import pallas as pl
from jax.experimental.pallas import tpu as pltpu
from jax.experimental.pallas import tpu_sc as plsc

N = 10000
E = 160000
D = 256
HD = D // 2
N2 = 10240
RT = N2 // 16
NA = 10112
RTA = NA // 16
CH = 128
CPT = 80
E2 = 16 * CPT * CH
EPT = CPT * CH
ER = E2 // CH
BR = N2 // 8
BRF = NA // 8

_mesh = plsc.VectorSubcoreMesh(core_axis_name="c", subcore_axis_name="s")


@functools.partial(
    pl.kernel,
    out_type=(
        jax.ShapeDtypeStruct((N2,), jnp.float32),
        jax.ShapeDtypeStruct((N2,), jnp.float32),
    ),
    mesh=_mesh,
    scratch_types=[
        pltpu.VMEM((CPT, CH), jnp.int32),
        pltpu.VMEM((CH,), jnp.float32),
        pltpu.VMEM((CH,), jnp.float32),
        pltpu.VMEM_SHARED((N2,), jnp.float32),
        pltpu.SemaphoreType.DMA,
    ],
)
def _degrees(dst_hbm, src_hbm, din_hbm, dout_hbm, idx_all, ones_v, zb_v, acc_sh,
             dsem):
    c = lax.axis_index("c")
    s = lax.axis_index("s")
    for i in range(CH // 16):
        ones_v[pl.ds(i * 16, 16)] = jnp.full((16,), 1.0, jnp.float32)
        zb_v[pl.ds(i * 16, 16)] = jnp.zeros((16,), jnp.float32)
    for k in range(RT // CH):
        pltpu.sync_copy(zb_v, acc_sh.at[pl.ds(s * RT + k * CH, CH)])

    @pl.when(c == 0)
    def _():
        pltpu.sync_copy(dst_hbm.at[pl.ds(s * CPT, CPT)], idx_all)

    @pl.when(c == 1)
    def _():
        pltpu.sync_copy(src_hbm.at[pl.ds(s * CPT, CPT)], idx_all)

    plsc.subcore_barrier()

    def chunk(k, u):
        pltpu.async_copy(ones_v, acc_sh.at[idx_all.at[k]], dsem, add=True)
        return u

    def drain(k, u):
        pltpu.make_async_copy(ones_v, acc_sh.at[idx_all.at[k]], dsem).wait()
        return u

    lax.fori_loop(0, CPT, chunk, 0)
    lax.fori_loop(0, CPT, drain, 0)
    plsc.subcore_barrier()

    @pl.when(c == 0)
    def _():
        pltpu.sync_copy(acc_sh.at[pl.ds(s * RT, RT)], din_hbm.at[pl.ds(s * RT, RT)])

    @pl.when(c == 1)
    def _():
        pltpu.sync_copy(acc_sh.at[pl.ds(s * RT, RT)], dout_hbm.at[pl.ds(s * RT, RT)])


@functools.partial(
    pl.kernel,
    out_type=(
        jax.ShapeDtypeStruct((NA, HD), jnp.float32),
        jax.ShapeDtypeStruct((NA, HD), jnp.float32),
    ),
    mesh=_mesh,
    scratch_types=[
        pltpu.VMEM((3, CH), jnp.int32),
        pltpu.VMEM((4, CH), jnp.int32),
        pltpu.VMEM((CH, HD), jnp.float32),
        pltpu.VMEM((CH, HD), jnp.float32),
        pltpu.VMEM((CH, HD), jnp.float32),
        pltpu.VMEM_SHARED((NA, HD), jnp.float32),
        pltpu.SemaphoreType.DMA,
        pltpu.SemaphoreType.DMA,
        pltpu.SemaphoreType.DMA,
        pltpu.SemaphoreType.DMA,
        pltpu.SemaphoreType.DMA,
        pltpu.SemaphoreType.DMA,
    ],
)
def _aggregate(xlo_hbm, xhi_hbm, srcf_hbm, dstf_hbm, h0_hbm, h1_hbm,
               src_v, dst_v, rows_0, rows_1, rows_2, acc_sh,
               gsem_0, gsem_1, gsem_2, isem_0, isem_1, isem_2):
    c = lax.axis_index("c")
    s = lax.axis_index("s")
    rows = (rows_0, rows_1, rows_2)
    gsem = (gsem_0, gsem_1, gsem_2)
    isem = (isem_0, isem_1, isem_2)

    def run(x_hbm):
        def idx_start(k, ps, qd):
            e0 = (s * CPT + k) * CH
            pltpu.async_copy(srcf_hbm.at[pl.ds(e0, CH)], src_v.at[ps], isem[ps])
            pltpu.async_copy(dstf_hbm.at[pl.ds(e0, CH)], dst_v.at[qd], isem[ps])

        def idx_wait(k, ps, qd):
            e0 = (s * CPT + k) * CH
            pltpu.make_async_copy(srcf_hbm.at[pl.ds(e0, CH)], src_v.at[ps],
                                  isem[ps]).wait()
            pltpu.make_async_copy(dstf_hbm.at[pl.ds(e0, CH)], dst_v.at[qd],
                                  isem[ps]).wait()

        def gather_start(p):
            pltpu.async_copy(x_hbm.at[src_v.at[p]], rows[p], gsem[p])

        def gather_wait(p):
            pltpu.make_async_copy(x_hbm.at[src_v.at[p]], rows[p],
                                  gsem[p]).wait()

        def scatter_start(p, q):
            pltpu.async_copy(rows[p], acc_sh.at[dst_v.at[q]], gsem[p],
                             add=True)

        def scatter_wait(p, q):
            pltpu.make_async_copy(rows[p], acc_sh.at[dst_v.at[q]],
                                  gsem[p]).wait()

        def process(k, m, wait_prev, prefetch, start_next):
            pr, qd = m % 3, m % 4
            pm3, pm4 = (m - 1) % 3, (m - 1) % 4
            gather_wait(pr)
            scatter_start(pr, qd)
            if wait_prev == "traced":
                @pl.when(k > 0)
                def _():
                    scatter_wait(pm3, pm4)
            elif wait_prev:
                scatter_wait(pm3, pm4)
            if prefetch:
                idx_wait(k + 2, (m + 2) % 3, (m + 2) % 4)
                gather_start((m + 2) % 3)
            if start_next:
                idx_start(k + 3, (m + 3) % 3, (m + 3) % 4)

        idx_start(0, 0, 0)
        idx_start(1, 1, 1)
        idx_start(2, 2, 2)

        def zrow(r, u):
            for j in range(HD // 16):
                rows_1[r, pl.ds(j * 16, 16)] = jnp.zeros((16,), jnp.float32)
            return u

        lax.fori_loop(0, CH, zrow, 0)
        idx_wait(0, 0, 0)
        gather_start(0)
        for k in range(RTA // CH):
            pltpu.sync_copy(rows_1, acc_sh.at[pl.ds(s * RTA + k * CH, CH)])
        pltpu.sync_copy(rows_1.at[pl.ds(0, RTA - (RTA // CH) * CH)],
                        acc_sh.at[pl.ds(s * RTA + (RTA // CH) * CH,
                                        RTA - (RTA // CH) * CH)])
        idx_wait(1, 1, 1)
        gather_start(1)
        plsc.subcore_barrier()

        def twelve(j, u):
            k0 = 12 * j
            for m in range(12):
                process(k0 + m, m, "traced" if m == 0 else True, True, True)
            return u

        lax.fori_loop(0, CPT // 12, twelve, 0)
        for k in range(12 * (CPT // 12), CPT):
            m = k % 12
            process(k, m, True, k + 2 < CPT, k + 3 < CPT)
        scatter_wait((CPT - 1) % 3, (CPT - 1) % 4)

    @pl.when(c == 0)
    def _():
        run(xlo_hbm)

    @pl.when(c == 1)
    def _():
        run(xhi_hbm)

    plsc.subcore_barrier()

    @pl.when(c == 0)
    def _():
        pltpu.sync_copy(acc_sh.at[pl.ds(s * RTA, RTA)], h0_hbm.at[pl.ds(s * RTA, RTA)])

    @pl.when(c == 1)
    def _():
        pltpu.sync_copy(acc_sh.at[pl.ds(s * RTA, RTA)], h1_hbm.at[pl.ds(s * RTA, RTA)])


def _edgeprep_body(ei_ref, srcf_ref, dstf_ref):
    pad = N + (lax.broadcasted_iota(jnp.int32, (E2 - E,), 0) % (NA - N))
    srcf_ref[...] = jnp.concatenate([ei_ref[0, :], pad])
    dstf_ref[...] = jnp.concatenate([ei_ref[1, :], pad])


def _edgeprep(edge_index):
    return pl.pallas_call(
        _edgeprep_body,
        grid=(1,),
        in_specs=[pl.BlockSpec((2, E), lambda i: (0, 0))],
        out_specs=(
            pl.BlockSpec((E2,), lambda i: (0,)),
            pl.BlockSpec((E2,), lambda i: (0,)),
        ),
        out_shape=(
            jax.ShapeDtypeStruct((E2,), jnp.int32),
            jax.ShapeDtypeStruct((E2,), jnp.int32),
        ),
    )(edge_index)


def _dense_body(prev_ref, raw_ref, w_ref, din_ref, dout_ref, xlo_ref, xhi_ref):
    i = pl.program_id(0)
    n_in = lax.rsqrt(jnp.maximum(din_ref[pl.ds(i * BR, BR)], 1.0))[:, None]
    n_out = lax.rsqrt(jnp.maximum(dout_ref[pl.ds(i * BR, BR)], 1.0))[:, None]
    p = jnp.dot(prev_ref[...], w_ref[...], preferred_element_type=jnp.float32)
    x = raw_ref[...] * n_in + p * n_out
    xlo_ref[...] = x[:, :HD]
    xhi_ref[...] = x[:, HD:]


def _dense(prev, raw, W, din2, dout2):
    return pl.pallas_call(
        _dense_body,
        grid=(N2 // BR,),
        in_specs=[
            pl.BlockSpec((BR, D), lambda i: (i, 0)),
            pl.BlockSpec((BR, D), lambda i: (i, 0)),
            pl.BlockSpec((D, D), lambda i: (0, 0)),
            pl.BlockSpec((N2,), lambda i: (0,)),
            pl.BlockSpec((N2,), lambda i: (0,)),
        ],
        out_specs=(
            pl.BlockSpec((BR, HD), lambda i: (i, 0)),
            pl.BlockSpec((BR, HD), lambda i: (i, 0)),
        ),
        out_shape=(
            jax.ShapeDtypeStruct((N2, HD), jnp.float32),
            jax.ShapeDtypeStruct((N2, HD), jnp.float32),
        ),
    )(prev, raw, W, din2, dout2)


def _final_body(h0_ref, h1_ref, din_ref, b_ref, o_ref):
    i = pl.program_id(0)
    n_in = lax.rsqrt(jnp.maximum(din_ref[pl.ds(i * BR, BR)], 1.0))[:, None]
    h = jnp.concatenate([h0_ref[...], h1_ref[...]], axis=1)
    o_ref[...] = jnp.maximum(h * n_in + b_ref[...], 0.0)


def _final(h0, h1, din2, b2):
    return pl.pallas_call(
        _final_body,
        grid=(N2 // BR,),
        in_specs=[
            pl.BlockSpec((BR, HD), lambda i: (i, 0)),
            pl.BlockSpec((BR, HD), lambda i: (i, 0)),
            pl.BlockSpec((N2,), lambda i: (0,)),
            pl.BlockSpec((1, D), lambda i: (0, 0)),
        ],
        out_specs=pl.BlockSpec((BR, D), lambda i: (i, 0)),
        out_shape=jax.ShapeDtypeStruct((N, D), jnp.float32),
    )(h0, h1, din2, b2)


def kernel(prev, raw, edge_index, W, b):
    srcf, dstf = _edgeprep(edge_index)
    src2d = srcf.reshape(ER, CH)
    dst2d = dstf.reshape(ER, CH)
    deg_in, deg_out = _degrees(dst2d, src2d)

    xlo, xhi = _dense(prev, raw, W, deg_in, deg_out)
    h0, h1 = _aggregate(xlo, xhi, srcf, dstf)
    return _final(h0, h1, deg_in, b.reshape(1, D))
